# Initial kernel scaffold; baseline (speedup 1.0000x reference)
import jax
import jax.numpy as jnp
from jax.experimental import pallas as pl


def kernel(x):
    raise NotImplementedError("write your pallas kernel here")


# speedup vs baseline: 1.2520x; 1.2520x over previous
import jax
import jax.numpy as jnp
from jax.experimental import pallas as pl
from jax.experimental.pallas import tpu as pltpu

_BR = 64
_C = 128
_L = 128


def _shift_right(x, d, axis):
    pad = [(0, 0)] * x.ndim
    pad[axis] = (d, 0)
    return jnp.pad(x, pad)[tuple(
        slice(0, x.shape[a]) if a == axis else slice(None)
        for a in range(x.ndim))]


def _body(x_ref, o_ref):
    x = x_ref[...]
    d = 1
    while d < _L:
        x = x + _shift_right(x, d, 2)
        d *= 2
    t = x[:, :, _L - 1:_L]
    c = t
    d = 1
    while d < _C:
        c = c + _shift_right(c, d, 1)
        d *= 2
    o_ref[...] = x + (c - t)


def kernel(x):
    n, m = x.shape
    xr = x.reshape(n, _C, _L)
    out = pl.pallas_call(
        _body,
        grid=(n // _BR,),
        in_specs=[pl.BlockSpec((_BR, _C, _L), lambda i: (i, 0, 0))],
        out_specs=pl.BlockSpec((_BR, _C, _L), lambda i: (i, 0, 0)),
        out_shape=jax.ShapeDtypeStruct((n, _C, _L), x.dtype),
        compiler_params=pltpu.CompilerParams(
            dimension_semantics=("arbitrary",),
            vmem_limit_bytes=100 * 1024 * 1024,
        ),
    )(xr)
    return out.reshape(n, m)


# speedup vs baseline: 1.6945x; 1.3535x over previous
import jax
import jax.numpy as jnp
from jax.experimental import pallas as pl
from jax.experimental.pallas import tpu as pltpu

_BR = 64
_C = 128
_L = 128


def _shift_right(x, d, axis):
    pad = [(0, 0)] * x.ndim
    pad[axis] = (d, 0)
    return jnp.pad(x, pad)[tuple(
        slice(0, x.shape[a]) if a == axis else slice(None)
        for a in range(x.ndim))]


def _body(x_ref, o_ref):
    x = x_ref[...]
    row = jax.lax.broadcasted_iota(jnp.int32, (_L, _L), 0)
    col = jax.lax.broadcasted_iota(jnp.int32, (_L, _L), 1)
    u = (row <= col).astype(jnp.bfloat16)
    xb = x.astype(jnp.bfloat16).reshape(_BR * _C, _L)
    y = jax.lax.dot(
        xb, u, preferred_element_type=jnp.float32
    ).reshape(_BR, _C, _L)
    t = y[:, :, _L - 1:_L]
    c = t
    d = 1
    while d < _C:
        c = c + _shift_right(c, d, 1)
        d *= 2
    o_ref[...] = y + (c - t)


def kernel(x):
    n, m = x.shape
    xr = x.reshape(n, _C, _L)
    out = pl.pallas_call(
        _body,
        grid=(n // _BR,),
        in_specs=[pl.BlockSpec((_BR, _C, _L), lambda i: (i, 0, 0))],
        out_specs=pl.BlockSpec((_BR, _C, _L), lambda i: (i, 0, 0)),
        out_shape=jax.ShapeDtypeStruct((n, _C, _L), x.dtype),
        compiler_params=pltpu.CompilerParams(
            dimension_semantics=("arbitrary",),
            vmem_limit_bytes=100 * 1024 * 1024,
        ),
    )(xr)
    return out.reshape(n, m)


# speedup vs baseline: 6.9951x; 4.1280x over previous
import jax
import jax.numpy as jnp
from jax.experimental import pallas as pl
from jax.experimental.pallas import tpu as pltpu

_BR = 512
_LB = 512
_G = 128


def _body(x_ref, w_ref, o_ref, carry_ref):
    j = pl.program_id(1)

    @pl.when(j == 0)
    def _():
        carry_ref[...] = jnp.zeros_like(carry_ref)

    xb = x_ref[...].astype(jnp.bfloat16)
    y2 = jax.lax.dot(xb, w_ref[...], preferred_element_type=jnp.float32)
    carry = carry_ref[...]
    o_ref[...] = y2[:, :_LB] + jnp.tile(carry, (1, _LB // _G))
    carry_ref[...] = carry + y2[:, _LB:]


def kernel(x):
    n, m = x.shape
    k_i = jax.lax.broadcasted_iota(jnp.int32, (_LB, _LB + _G), 0)
    n_i = jax.lax.broadcasted_iota(jnp.int32, (_LB, _LB + _G), 1)
    w = ((k_i <= n_i) | (n_i >= _LB)).astype(jnp.bfloat16)
    return pl.pallas_call(
        _body,
        grid=(n // _BR, m // _LB),
        in_specs=[
            pl.BlockSpec((_BR, _LB), lambda i, j: (i, j)),
            pl.BlockSpec((_LB, _LB + _G), lambda i, j: (0, 0)),
        ],
        out_specs=pl.BlockSpec((_BR, _LB), lambda i, j: (i, j)),
        out_shape=jax.ShapeDtypeStruct((n, m), x.dtype),
        scratch_shapes=[pltpu.VMEM((_BR, _G), jnp.float32)],
        compiler_params=pltpu.CompilerParams(
            dimension_semantics=("arbitrary", "arbitrary"),
        ),
    )(x, w)
